# baseline (speedup 1.0000x reference)
import functools

import jax
import jax.numpy as jnp
from jax import lax
from jax.experimental import pallas as pl
from jax.experimental.pallas import tpu as pltpu
from jax.experimental.pallas import tpu_sc as plsc

_NC = 2
_NS = 16
_CHUNK = 128
_ACHUNK = 128
_FRAC0 = 0.72


def _sc_mesh():
    return plsc.VectorSubcoreMesh(core_axis_name="c", subcore_axis_name="s")


def _degrees_call(deg_idx, n_pad):
    rows_half = deg_idx.shape[0] // 2
    rows_per_tile = rows_half // _NS
    n_per_tile = n_pad // _NS
    _LAG = 8

    @functools.partial(
        pl.kernel,
        out_type=jax.ShapeDtypeStruct((2, n_pad, 16), jnp.float32),
        mesh=_sc_mesh(),
        scratch_types=[
            pltpu.VMEM((rows_per_tile, _CHUNK), jnp.int32),
            pltpu.VMEM((_CHUNK, 16), jnp.float32),
            pltpu.VMEM_SHARED((n_pad, 16), jnp.float32),
            pltpu.SemaphoreType.DMA,
        ],
        compiler_params=pltpu.CompilerParams(use_tc_tiling_on_sc=False),
    )
    def deg_kernel(idx_hbm, ones_hbm, zeros_hbm, out_hbm,
                   idx_v, ones_v, acc, ssem):
        cid = lax.axis_index("c")
        sid = lax.axis_index("s")
        pltpu.sync_copy(ones_hbm, ones_v)
        pltpu.sync_copy(
            idx_hbm.at[pl.ds(cid * rows_half + sid * rows_per_tile,
                             rows_per_tile)], idx_v)
        pltpu.sync_copy(zeros_hbm, acc.at[pl.ds(sid * n_per_tile, n_per_tile)])
        plsc.subcore_barrier()

        def drain(_i):
            pltpu.make_async_copy(ones_v, acc.at[idx_v.at[0]], ssem).wait()

        def body(i, _):
            pltpu.async_copy(ones_v, acc.at[idx_v.at[i]], ssem, add=True)

            @pl.when(i >= _LAG)
            def _():
                drain(i)
            return 0

        lax.fori_loop(0, rows_per_tile, body, 0)
        lax.fori_loop(0, _LAG, lambda i, _: (drain(i), 0)[1], 0)
        plsc.subcore_barrier()
        pltpu.sync_copy(
            acc.at[pl.ds(sid * n_per_tile, n_per_tile)],
            out_hbm.at[cid, pl.ds(sid * n_per_tile, n_per_tile)])

    return deg_kernel(
        deg_idx,
        jnp.ones((_CHUNK, 16), jnp.float32),
        jnp.zeros((n_per_tile, 16), jnp.float32),
    )


def _aggregate_call(x, src_p, dst_p, n_pad, frac0=0.5):
    d = x.shape[1]
    rows_total = src_p.shape[0]
    ch = src_p.shape[1]
    blk = 16
    rows_pair = rows_total // _NS
    r0 = int(round(rows_pair * frac0 / blk)) * blk
    r1 = rows_pair - r0
    n_per_tile = n_pad // _NS
    nbuf = 2

    @functools.partial(
        pl.kernel,
        out_type=jax.ShapeDtypeStruct((2, n_pad, d), jnp.float32),
        mesh=_sc_mesh(),
        scratch_types=(
            [pltpu.VMEM((blk, ch), jnp.int32)] * 2
            + [pltpu.VMEM((ch, d), jnp.float32)] * nbuf
            + [pltpu.VMEM_SHARED((n_pad, d), jnp.float32)]
            + [pltpu.SemaphoreType.DMA] * (2 * nbuf)
        ),
        compiler_params=pltpu.CompilerParams(use_tc_tiling_on_sc=False),
    )
    def agg_kernel(x_hbm, src_hbm, dst_hbm, zeros_hbm, out_hbm,
                   sidx_v, didx_v, *rest):
        bufs = rest[:nbuf]
        acc = rest[nbuf]
        gsems = rest[nbuf + 1:2 * nbuf + 1]
        ssems = rest[2 * nbuf + 1:]
        cid = lax.axis_index("c")
        sid = lax.axis_index("s")
        row0 = jnp.where(cid == 0, sid * r0, _NS * r0 + sid * r1)
        my_blocks = jnp.where(cid == 0, r0 // blk, r1 // blk)
        pltpu.sync_copy(zeros_hbm, acc.at[pl.ds(sid * n_per_tile, n_per_tile)])
        plsc.subcore_barrier()

        def block(o, _):
            pltpu.sync_copy(src_hbm.at[pl.ds(row0 + o * blk, blk)], sidx_v)
            pltpu.sync_copy(dst_hbm.at[pl.ds(row0 + o * blk, blk)], didx_v)
            for k in range(nbuf - 1):
                pltpu.async_copy(x_hbm.at[sidx_v.at[k]], bufs[k], gsems[k])
            for k in range(blk):
                b = k % nbuf
                pltpu.make_async_copy(x_hbm.at[sidx_v.at[k]],
                                      bufs[b], gsems[b]).wait()
                pltpu.async_copy(bufs[b], acc.at[didx_v.at[k]], ssems[b],
                                 add=True)
                kn = k + nbuf - 1
                if kn < blk:
                    bn = kn % nbuf
                    if k >= 1:
                        pltpu.make_async_copy(
                            bufs[bn], acc.at[didx_v.at[k - 1]],
                            ssems[bn]).wait()
                    pltpu.async_copy(x_hbm.at[sidx_v.at[kn]],
                                     bufs[bn], gsems[bn])
            for k in range(blk - nbuf, blk):
                b = k % nbuf
                pltpu.make_async_copy(bufs[b], acc.at[didx_v.at[k]],
                                      ssems[b]).wait()
            return 0

        lax.fori_loop(0, my_blocks, block, 0)
        plsc.subcore_barrier()
        pltpu.sync_copy(
            acc.at[pl.ds(sid * n_per_tile, n_per_tile)],
            out_hbm.at[cid, pl.ds(sid * n_per_tile, n_per_tile)])

    return agg_kernel(x, src_p, dst_p,
                      jnp.zeros((n_per_tile, d), jnp.float32))


_ROWS = 1000


def _rs(deg_block):
    return lax.rsqrt(jnp.maximum(deg_block[:, 0:1], 1.0))


def _scale_body(x_ref, degs_ref, o_ref):
    o_ref[...] = x_ref[...] * _rs(degs_ref[0])


def _layer1_body(aggp_ref, degs_ref, w1_ref, b1_ref, w2_ref, o_ref):
    agg = aggp_ref[0] + aggp_ref[1]
    t = agg * _rs(degs_ref[1])
    t = jnp.dot(t, w1_ref[...], preferred_element_type=jnp.float32)
    t = jnp.maximum(t + b1_ref[...], 0.0)
    t = t * _rs(degs_ref[0])
    o_ref[...] = jnp.dot(t, w2_ref[...], preferred_element_type=jnp.float32)


def _layer2_body(aggp_ref, degs_ref, b2_ref, w3_ref, o_ref):
    agg = aggp_ref[0] + aggp_ref[1]
    t = jnp.maximum(agg * _rs(degs_ref[1]) + b2_ref[...], 0.0)
    t = t * _rs(degs_ref[0])
    o_ref[...] = jnp.dot(t, w3_ref[...], preferred_element_type=jnp.float32)


def _layer3_body(aggp_ref, degs_ref, b3_ref, o_ref):
    c = o_ref.shape[1]
    agg = aggp_ref[0][:, :c] + aggp_ref[1][:, :c]
    o_ref[...] = agg * _rs(degs_ref[1]) + b3_ref[...]


def _full(shape):
    return pl.BlockSpec(shape, lambda i: (0,) * len(shape))


def _rows_spec(d):
    return pl.BlockSpec((_ROWS, d), lambda i: (i, 0))


def _stack_spec(d):
    return pl.BlockSpec((2, _ROWS, d), lambda i: (0, i, 0))


def kernel(in_feat, edge_index, W1, b1, W2, b2, W3, b3):
    n, d_in = in_feat.shape
    h = W1.shape[1]
    hm = W2.shape[1]
    c = W3.shape[1]
    e = edge_index.shape[1]
    grid = (n // _ROWS,)

    e_unit = _CHUNK * _NC * _NS * 8
    e_pad = ((e + e_unit - 1) // e_unit) * e_unit
    n_pad = ((n + _NS * 8) // (_NS * 8)) * (_NS * 8)

    src = edge_index[0]
    dst = edge_index[1]
    pad = e_pad - e
    dummy = n + jnp.arange(pad, dtype=jnp.int32) % (n_pad - n)
    src_g = jnp.concatenate([src, jnp.zeros((pad,), jnp.int32)])
    src_d = jnp.concatenate([src, dummy])
    dst_d = jnp.concatenate([dst, dummy])
    deg_idx = jnp.concatenate([src_d, dst_d]).reshape(-1, _CHUNK)
    src_g = src_g.reshape(-1, _ACHUNK)
    dst_d = dst_d.reshape(-1, _ACHUNK)

    degs = _degrees_call(deg_idx, n_pad)

    x0 = pl.pallas_call(
        _scale_body,
        grid=grid,
        in_specs=[_rows_spec(d_in), _stack_spec(16)],
        out_specs=_rows_spec(d_in),
        out_shape=jax.ShapeDtypeStruct((n, d_in), jnp.float32),
    )(in_feat, degs)

    agg1 = _aggregate_call(x0, src_g, dst_d, n_pad, _FRAC0)

    y2 = pl.pallas_call(
        _layer1_body,
        grid=grid,
        in_specs=[_stack_spec(d_in), _stack_spec(16), _full((d_in, h)),
                  _full((1, h)), _full((h, hm))],
        out_specs=_rows_spec(hm),
        out_shape=jax.ShapeDtypeStruct((n, hm), jnp.float32),
    )(agg1, degs, W1, b1.reshape(1, h), W2)

    agg2 = _aggregate_call(y2, src_g, dst_d, n_pad, _FRAC0)

    c_pad = c
    w3p = jnp.concatenate([W3, jnp.zeros((hm, c_pad - c), jnp.float32)], 1)
    y3 = pl.pallas_call(
        _layer2_body,
        grid=grid,
        in_specs=[_stack_spec(hm), _stack_spec(16), _full((1, hm)),
                  _full((hm, c_pad))],
        out_specs=_rows_spec(c_pad),
        out_shape=jax.ShapeDtypeStruct((n, c_pad), jnp.float32),
    )(agg2, degs, b2.reshape(1, hm), w3p)

    agg3 = _aggregate_call(y3, src_g, dst_d, n_pad, _FRAC0)

    out = pl.pallas_call(
        _layer3_body,
        grid=grid,
        in_specs=[_stack_spec(c_pad), _stack_spec(16), _full((1, c))],
        out_specs=_rows_spec(c),
        out_shape=jax.ShapeDtypeStruct((n, c), jnp.float32),
    )(agg3, degs, b3.reshape(1, c))

    return out


# speedup vs baseline: 1.0122x; 1.0122x over previous
import functools

import jax
import jax.numpy as jnp
from jax import lax
from jax.experimental import pallas as pl
from jax.experimental.pallas import tpu as pltpu
from jax.experimental.pallas import tpu_sc as plsc

_NC = 2
_NS = 16
_CHUNK = 128
_ACHUNK = 128
_FRAC0 = 0.85


def _sc_mesh():
    return plsc.VectorSubcoreMesh(core_axis_name="c", subcore_axis_name="s")


def _degrees_call(deg_idx, n_pad):
    rows_half = deg_idx.shape[0] // 2
    rows_per_tile = rows_half // _NS
    n_per_tile = n_pad // _NS
    _LAG = 8

    @functools.partial(
        pl.kernel,
        out_type=jax.ShapeDtypeStruct((2, n_pad, 16), jnp.float32),
        mesh=_sc_mesh(),
        scratch_types=[
            pltpu.VMEM((rows_per_tile, _CHUNK), jnp.int32),
            pltpu.VMEM((_CHUNK, 16), jnp.float32),
            pltpu.VMEM_SHARED((n_pad, 16), jnp.float32),
            pltpu.SemaphoreType.DMA,
        ],
        compiler_params=pltpu.CompilerParams(use_tc_tiling_on_sc=False),
    )
    def deg_kernel(idx_hbm, ones_hbm, zeros_hbm, out_hbm,
                   idx_v, ones_v, acc, ssem):
        cid = lax.axis_index("c")
        sid = lax.axis_index("s")
        pltpu.sync_copy(ones_hbm, ones_v)
        pltpu.sync_copy(
            idx_hbm.at[pl.ds(cid * rows_half + sid * rows_per_tile,
                             rows_per_tile)], idx_v)
        pltpu.sync_copy(zeros_hbm, acc.at[pl.ds(sid * n_per_tile, n_per_tile)])
        plsc.subcore_barrier()

        def drain(_i):
            pltpu.make_async_copy(ones_v, acc.at[idx_v.at[0]], ssem).wait()

        def body(i, _):
            pltpu.async_copy(ones_v, acc.at[idx_v.at[i]], ssem, add=True)

            @pl.when(i >= _LAG)
            def _():
                drain(i)
            return 0

        lax.fori_loop(0, rows_per_tile, body, 0)
        lax.fori_loop(0, _LAG, lambda i, _: (drain(i), 0)[1], 0)
        plsc.subcore_barrier()
        pltpu.sync_copy(
            acc.at[pl.ds(sid * n_per_tile, n_per_tile)],
            out_hbm.at[cid, pl.ds(sid * n_per_tile, n_per_tile)])

    return deg_kernel(
        deg_idx,
        jnp.ones((_CHUNK, 16), jnp.float32),
        jnp.zeros((n_per_tile, 16), jnp.float32),
    )


def _aggregate_call(x, src_p, dst_p, n_pad, frac0=0.5):
    d = x.shape[1]
    rows_total = src_p.shape[0]
    ch = src_p.shape[1]
    blk = 16
    rows_pair = rows_total // _NS
    r0 = int(round(rows_pair * frac0 / blk)) * blk
    r1 = rows_pair - r0
    n_per_tile = n_pad // _NS
    nbuf = 2

    @functools.partial(
        pl.kernel,
        out_type=jax.ShapeDtypeStruct((2, n_pad, d), jnp.float32),
        mesh=_sc_mesh(),
        scratch_types=(
            [pltpu.VMEM((blk, ch), jnp.int32)] * 2
            + [pltpu.VMEM((ch, d), jnp.float32)] * nbuf
            + [pltpu.VMEM_SHARED((n_pad, d), jnp.float32)]
            + [pltpu.SemaphoreType.DMA] * (2 * nbuf)
        ),
        compiler_params=pltpu.CompilerParams(use_tc_tiling_on_sc=False),
    )
    def agg_kernel(x_hbm, src_hbm, dst_hbm, zeros_hbm, out_hbm,
                   sidx_v, didx_v, *rest):
        bufs = rest[:nbuf]
        acc = rest[nbuf]
        gsems = rest[nbuf + 1:2 * nbuf + 1]
        ssems = rest[2 * nbuf + 1:]
        cid = lax.axis_index("c")
        sid = lax.axis_index("s")
        row0 = jnp.where(cid == 0, sid * r0, _NS * r0 + sid * r1)
        my_blocks = jnp.where(cid == 0, r0 // blk, r1 // blk)
        pltpu.sync_copy(zeros_hbm, acc.at[pl.ds(sid * n_per_tile, n_per_tile)])
        plsc.subcore_barrier()

        def block(o, _):
            pltpu.sync_copy(src_hbm.at[pl.ds(row0 + o * blk, blk)], sidx_v)
            pltpu.sync_copy(dst_hbm.at[pl.ds(row0 + o * blk, blk)], didx_v)
            for k in range(nbuf - 1):
                pltpu.async_copy(x_hbm.at[sidx_v.at[k]], bufs[k], gsems[k])
            for k in range(blk):
                b = k % nbuf
                pltpu.make_async_copy(x_hbm.at[sidx_v.at[k]],
                                      bufs[b], gsems[b]).wait()
                pltpu.async_copy(bufs[b], acc.at[didx_v.at[k]], ssems[b],
                                 add=True)
                kn = k + nbuf - 1
                if kn < blk:
                    bn = kn % nbuf
                    if k >= 1:
                        pltpu.make_async_copy(
                            bufs[bn], acc.at[didx_v.at[k - 1]],
                            ssems[bn]).wait()
                    pltpu.async_copy(x_hbm.at[sidx_v.at[kn]],
                                     bufs[bn], gsems[bn])
            for k in range(blk - nbuf, blk):
                b = k % nbuf
                pltpu.make_async_copy(bufs[b], acc.at[didx_v.at[k]],
                                      ssems[b]).wait()
            return 0

        lax.fori_loop(0, my_blocks, block, 0)
        plsc.subcore_barrier()
        pltpu.sync_copy(
            acc.at[pl.ds(sid * n_per_tile, n_per_tile)],
            out_hbm.at[cid, pl.ds(sid * n_per_tile, n_per_tile)])

    return agg_kernel(x, src_p, dst_p,
                      jnp.zeros((n_per_tile, d), jnp.float32))


_ROWS = 1000


def _rs(deg_block):
    return lax.rsqrt(jnp.maximum(deg_block[:, 0:1], 1.0))


def _scale_body(x_ref, degs_ref, o_ref):
    o_ref[...] = x_ref[...] * _rs(degs_ref[0])


def _layer1_body(aggp_ref, degs_ref, w1_ref, b1_ref, w2_ref, o_ref):
    agg = aggp_ref[0] + aggp_ref[1]
    t = agg * _rs(degs_ref[1])
    t = jnp.dot(t, w1_ref[...], preferred_element_type=jnp.float32)
    t = jnp.maximum(t + b1_ref[...], 0.0)
    t = t * _rs(degs_ref[0])
    o_ref[...] = jnp.dot(t, w2_ref[...], preferred_element_type=jnp.float32)


def _layer2_body(aggp_ref, degs_ref, b2_ref, w3_ref, o_ref):
    agg = aggp_ref[0] + aggp_ref[1]
    t = jnp.maximum(agg * _rs(degs_ref[1]) + b2_ref[...], 0.0)
    t = t * _rs(degs_ref[0])
    o_ref[...] = jnp.dot(t, w3_ref[...], preferred_element_type=jnp.float32)


def _layer3_body(aggp_ref, degs_ref, b3_ref, o_ref):
    c = o_ref.shape[1]
    agg = aggp_ref[0][:, :c] + aggp_ref[1][:, :c]
    o_ref[...] = agg * _rs(degs_ref[1]) + b3_ref[...]


def _full(shape):
    return pl.BlockSpec(shape, lambda i: (0,) * len(shape))


def _rows_spec(d):
    return pl.BlockSpec((_ROWS, d), lambda i: (i, 0))


def _stack_spec(d):
    return pl.BlockSpec((2, _ROWS, d), lambda i: (0, i, 0))


def kernel(in_feat, edge_index, W1, b1, W2, b2, W3, b3):
    n, d_in = in_feat.shape
    h = W1.shape[1]
    hm = W2.shape[1]
    c = W3.shape[1]
    e = edge_index.shape[1]
    grid = (n // _ROWS,)

    e_unit = _CHUNK * _NC * _NS * 8
    e_pad = ((e + e_unit - 1) // e_unit) * e_unit
    n_pad = ((n + _NS * 8) // (_NS * 8)) * (_NS * 8)

    src = edge_index[0]
    dst = edge_index[1]
    pad = e_pad - e
    dummy = n + jnp.arange(pad, dtype=jnp.int32) % (n_pad - n)
    src_g = jnp.concatenate([src, jnp.zeros((pad,), jnp.int32)])
    src_d = jnp.concatenate([src, dummy])
    dst_d = jnp.concatenate([dst, dummy])
    deg_idx = jnp.concatenate([src_d, dst_d]).reshape(-1, _CHUNK)
    src_g = src_g.reshape(-1, _ACHUNK)
    dst_d = dst_d.reshape(-1, _ACHUNK)

    degs = _degrees_call(deg_idx, n_pad)

    x0 = pl.pallas_call(
        _scale_body,
        grid=grid,
        in_specs=[_rows_spec(d_in), _stack_spec(16)],
        out_specs=_rows_spec(d_in),
        out_shape=jax.ShapeDtypeStruct((n, d_in), jnp.float32),
    )(in_feat, degs)

    agg1 = _aggregate_call(x0, src_g, dst_d, n_pad, _FRAC0)

    y2 = pl.pallas_call(
        _layer1_body,
        grid=grid,
        in_specs=[_stack_spec(d_in), _stack_spec(16), _full((d_in, h)),
                  _full((1, h)), _full((h, hm))],
        out_specs=_rows_spec(hm),
        out_shape=jax.ShapeDtypeStruct((n, hm), jnp.float32),
    )(agg1, degs, W1, b1.reshape(1, h), W2)

    agg2 = _aggregate_call(y2, src_g, dst_d, n_pad, _FRAC0)

    c_pad = c
    w3p = jnp.concatenate([W3, jnp.zeros((hm, c_pad - c), jnp.float32)], 1)
    y3 = pl.pallas_call(
        _layer2_body,
        grid=grid,
        in_specs=[_stack_spec(hm), _stack_spec(16), _full((1, hm)),
                  _full((hm, c_pad))],
        out_specs=_rows_spec(c_pad),
        out_shape=jax.ShapeDtypeStruct((n, c_pad), jnp.float32),
    )(agg2, degs, b2.reshape(1, hm), w3p)

    agg3 = _aggregate_call(y3, src_g, dst_d, n_pad, _FRAC0)

    out = pl.pallas_call(
        _layer3_body,
        grid=grid,
        in_specs=[_stack_spec(c_pad), _stack_spec(16), _full((1, c))],
        out_specs=_rows_spec(c),
        out_shape=jax.ShapeDtypeStruct((n, c), jnp.float32),
    )(agg3, degs, b3.reshape(1, c))

    return out


# speedup vs baseline: 1.0412x; 1.0287x over previous
import functools

import jax
import jax.numpy as jnp
from jax import lax
from jax.experimental import pallas as pl
from jax.experimental.pallas import tpu as pltpu
from jax.experimental.pallas import tpu_sc as plsc

_NC = 2
_NS = 16
_CHUNK = 128
_ACHUNK = 128
_FRAC0 = 0.85


def _sc_mesh():
    return plsc.VectorSubcoreMesh(core_axis_name="c", subcore_axis_name="s")


def _degrees_call(deg_idx, n_pad):
    rows_half = deg_idx.shape[0] // 2
    rows_per_tile = rows_half // _NS
    n_per_tile = n_pad // _NS
    _LAG = 8

    @functools.partial(
        pl.kernel,
        out_type=jax.ShapeDtypeStruct((2, n_pad, 16), jnp.float32),
        mesh=_sc_mesh(),
        scratch_types=[
            pltpu.VMEM((rows_per_tile, _CHUNK), jnp.int32),
            pltpu.VMEM((_CHUNK, 16), jnp.float32),
            pltpu.VMEM_SHARED((n_pad, 16), jnp.float32),
            pltpu.SemaphoreType.DMA,
        ],
        compiler_params=pltpu.CompilerParams(use_tc_tiling_on_sc=False),
    )
    def deg_kernel(idx_hbm, ones_hbm, zeros_hbm, out_hbm,
                   idx_v, ones_v, acc, ssem):
        cid = lax.axis_index("c")
        sid = lax.axis_index("s")
        pltpu.sync_copy(ones_hbm, ones_v)
        pltpu.sync_copy(
            idx_hbm.at[pl.ds(cid * rows_half + sid * rows_per_tile,
                             rows_per_tile)], idx_v)
        pltpu.sync_copy(zeros_hbm, acc.at[pl.ds(sid * n_per_tile, n_per_tile)])
        plsc.subcore_barrier()

        def drain(_i):
            pltpu.make_async_copy(ones_v, acc.at[idx_v.at[0]], ssem).wait()

        def body(i, _):
            pltpu.async_copy(ones_v, acc.at[idx_v.at[i]], ssem, add=True)

            @pl.when(i >= _LAG)
            def _():
                drain(i)
            return 0

        lax.fori_loop(0, rows_per_tile, body, 0)
        lax.fori_loop(0, _LAG, lambda i, _: (drain(i), 0)[1], 0)
        plsc.subcore_barrier()
        pltpu.sync_copy(
            acc.at[pl.ds(sid * n_per_tile, n_per_tile)],
            out_hbm.at[cid, pl.ds(sid * n_per_tile, n_per_tile)])

    return deg_kernel(
        deg_idx,
        jnp.ones((_CHUNK, 16), jnp.float32),
        jnp.zeros((n_per_tile, 16), jnp.float32),
    )


def _aggregate_call(x, src_p, dst_p, n_pad, frac0=0.5):
    d = x.shape[1]
    rows_total = src_p.shape[0]
    ch = src_p.shape[1]
    blk = 16
    rows_pair = rows_total // _NS
    r0 = int(round(rows_pair * frac0 / blk)) * blk
    r1 = rows_pair - r0
    n_per_tile = n_pad // _NS
    nbuf = 2

    @functools.partial(
        pl.kernel,
        out_type=jax.ShapeDtypeStruct((2, n_pad, d), jnp.float32),
        mesh=_sc_mesh(),
        scratch_types=(
            [pltpu.VMEM((blk, ch), jnp.int32)] * 2
            + [pltpu.VMEM((ch, d), jnp.float32)] * nbuf
            + [pltpu.VMEM_SHARED((n_pad, d), jnp.float32)]
            + [pltpu.SemaphoreType.DMA] * (2 * nbuf)
        ),
        compiler_params=pltpu.CompilerParams(use_tc_tiling_on_sc=False),
    )
    def agg_kernel(x_hbm, src_hbm, dst_hbm, out_hbm,
                   sidx_v, didx_v, *rest):
        bufs = rest[:nbuf]
        acc = rest[nbuf]
        gsems = rest[nbuf + 1:2 * nbuf + 1]
        ssems = rest[2 * nbuf + 1:]
        cid = lax.axis_index("c")
        sid = lax.axis_index("s")
        row0 = jnp.where(cid == 0, sid * r0, _NS * r0 + sid * r1)
        my_blocks = jnp.where(cid == 0, r0 // blk, r1 // blk)

        def zrow(i, _):
            for j in range(d // 16):
                bufs[0][i, pl.ds(j * 16, 16)] = jnp.zeros((16,), jnp.float32)
            return 0

        lax.fori_loop(0, ch, zrow, 0)
        zbase = sid * n_per_tile
        nfull = n_per_tile // ch
        for t in range(nfull):
            pltpu.sync_copy(bufs[0], acc.at[pl.ds(zbase + t * ch, ch)])
        rem = n_per_tile - nfull * ch
        if rem:
            pltpu.sync_copy(bufs[0].at[pl.ds(0, rem)],
                            acc.at[pl.ds(zbase + nfull * ch, rem)])
        plsc.subcore_barrier()

        def block(o, _):
            pltpu.sync_copy(src_hbm.at[pl.ds(row0 + o * blk, blk)], sidx_v)
            pltpu.sync_copy(dst_hbm.at[pl.ds(row0 + o * blk, blk)], didx_v)
            for k in range(nbuf - 1):
                pltpu.async_copy(x_hbm.at[sidx_v.at[k]], bufs[k], gsems[k])
            for k in range(blk):
                b = k % nbuf
                pltpu.make_async_copy(x_hbm.at[sidx_v.at[k]],
                                      bufs[b], gsems[b]).wait()
                pltpu.async_copy(bufs[b], acc.at[didx_v.at[k]], ssems[b],
                                 add=True)
                kn = k + nbuf - 1
                if kn < blk:
                    bn = kn % nbuf
                    if k >= 1:
                        pltpu.make_async_copy(
                            bufs[bn], acc.at[didx_v.at[k - 1]],
                            ssems[bn]).wait()
                    pltpu.async_copy(x_hbm.at[sidx_v.at[kn]],
                                     bufs[bn], gsems[bn])
            for k in range(blk - nbuf, blk):
                b = k % nbuf
                pltpu.make_async_copy(bufs[b], acc.at[didx_v.at[k]],
                                      ssems[b]).wait()
            return 0

        lax.fori_loop(0, my_blocks, block, 0)
        plsc.subcore_barrier()
        pltpu.sync_copy(
            acc.at[pl.ds(sid * n_per_tile, n_per_tile)],
            out_hbm.at[cid, pl.ds(sid * n_per_tile, n_per_tile)])

    return agg_kernel(x, src_p, dst_p)


_ROWS = 1000


def _rs(deg_block):
    return lax.rsqrt(jnp.maximum(deg_block[:, 0:1], 1.0))


def _scale_body(x_ref, degs_ref, o_ref):
    o_ref[...] = x_ref[...] * _rs(degs_ref[0])


def _layer1_body(aggp_ref, degs_ref, w1_ref, b1_ref, w2_ref, o_ref):
    agg = aggp_ref[0] + aggp_ref[1]
    t = agg * _rs(degs_ref[1])
    t = jnp.dot(t, w1_ref[...], preferred_element_type=jnp.float32)
    t = jnp.maximum(t + b1_ref[...], 0.0)
    t = t * _rs(degs_ref[0])
    o_ref[...] = jnp.dot(t, w2_ref[...], preferred_element_type=jnp.float32)


def _layer2_body(aggp_ref, degs_ref, b2_ref, w3_ref, o_ref):
    agg = aggp_ref[0] + aggp_ref[1]
    t = jnp.maximum(agg * _rs(degs_ref[1]) + b2_ref[...], 0.0)
    t = t * _rs(degs_ref[0])
    o_ref[...] = jnp.dot(t, w3_ref[...], preferred_element_type=jnp.float32)


def _layer3_body(aggp_ref, degs_ref, b3_ref, o_ref):
    c = o_ref.shape[1]
    agg = aggp_ref[0][:, :c] + aggp_ref[1][:, :c]
    o_ref[...] = agg * _rs(degs_ref[1]) + b3_ref[...]


def _full(shape):
    return pl.BlockSpec(shape, lambda i: (0,) * len(shape))


def _rows_spec(d):
    return pl.BlockSpec((_ROWS, d), lambda i: (i, 0))


def _stack_spec(d):
    return pl.BlockSpec((2, _ROWS, d), lambda i: (0, i, 0))


def kernel(in_feat, edge_index, W1, b1, W2, b2, W3, b3):
    n, d_in = in_feat.shape
    h = W1.shape[1]
    hm = W2.shape[1]
    c = W3.shape[1]
    e = edge_index.shape[1]
    grid = (n // _ROWS,)

    e_unit = _CHUNK * _NC * _NS * 8
    e_pad = ((e + e_unit - 1) // e_unit) * e_unit
    n_pad = ((n + _NS * 8) // (_NS * 8)) * (_NS * 8)

    src = edge_index[0]
    dst = edge_index[1]
    pad = e_pad - e
    dummy = n + jnp.arange(pad, dtype=jnp.int32) % (n_pad - n)
    src_g = jnp.concatenate([src, jnp.zeros((pad,), jnp.int32)])
    src_d = jnp.concatenate([src, dummy])
    dst_d = jnp.concatenate([dst, dummy])
    deg_idx = jnp.concatenate([src_d, dst_d]).reshape(-1, _CHUNK)
    src_g = src_g.reshape(-1, _ACHUNK)
    dst_d = dst_d.reshape(-1, _ACHUNK)

    degs = _degrees_call(deg_idx, n_pad)

    x0 = pl.pallas_call(
        _scale_body,
        grid=grid,
        in_specs=[_rows_spec(d_in), _stack_spec(16)],
        out_specs=_rows_spec(d_in),
        out_shape=jax.ShapeDtypeStruct((n, d_in), jnp.float32),
    )(in_feat, degs)

    agg1 = _aggregate_call(x0, src_g, dst_d, n_pad, _FRAC0)

    y2 = pl.pallas_call(
        _layer1_body,
        grid=grid,
        in_specs=[_stack_spec(d_in), _stack_spec(16), _full((d_in, h)),
                  _full((1, h)), _full((h, hm))],
        out_specs=_rows_spec(hm),
        out_shape=jax.ShapeDtypeStruct((n, hm), jnp.float32),
    )(agg1, degs, W1, b1.reshape(1, h), W2)

    agg2 = _aggregate_call(y2, src_g, dst_d, n_pad, _FRAC0)

    c_pad = c
    w3p = jnp.concatenate([W3, jnp.zeros((hm, c_pad - c), jnp.float32)], 1)
    y3 = pl.pallas_call(
        _layer2_body,
        grid=grid,
        in_specs=[_stack_spec(hm), _stack_spec(16), _full((1, hm)),
                  _full((hm, c_pad))],
        out_specs=_rows_spec(c_pad),
        out_shape=jax.ShapeDtypeStruct((n, c_pad), jnp.float32),
    )(agg2, degs, b2.reshape(1, hm), w3p)

    agg3 = _aggregate_call(y3, src_g, dst_d, n_pad, _FRAC0)

    out = pl.pallas_call(
        _layer3_body,
        grid=grid,
        in_specs=[_stack_spec(c_pad), _stack_spec(16), _full((1, c))],
        out_specs=_rows_spec(c),
        out_shape=jax.ShapeDtypeStruct((n, c), jnp.float32),
    )(agg3, degs, b3.reshape(1, c))

    return out
